# baseline (speedup 1.0000x reference)
import jax
import jax.numpy as jnp
from jax.experimental import pallas as pl
from jax.experimental.pallas import tpu as pltpu

SIZE = 1024
LATENT_DIM = 256
BETA_C = 0.25
N_ROWS = 4 * 8 * 32 * 32
TILE = 512
N_TILES = N_ROWS // TILE
TILES_PER_B = 8192 // TILE


def _vq_body(zb_ref, emb_ref, zq_ref, enc_ref, idx_ref, loss_ref, perp_ref,
             counts_ref, loss_acc_ref):
    t = pl.program_id(0)

    @pl.when(t == 0)
    def _init():
        counts_ref[...] = jnp.zeros_like(counts_ref)
        loss_acc_ref[0] = 0.0

    zb = zb_ref[0]
    e = emb_ref[...]
    en = jnp.sum(e * e, axis=1)

    ez = jax.lax.dot_general(e, zb, (((1,), (0,)), ((), ())),
                             preferred_element_type=jnp.float32)
    s = en[:, None] - 2.0 * ez

    minv = jnp.min(s, axis=0)
    code_iota = jax.lax.broadcasted_iota(jnp.int32, (SIZE, TILE), 0)
    idx = jnp.min(jnp.where(s == minv[None, :], code_iota, SIZE), axis=0)

    oh_T = (code_iota == idx[None, :]).astype(jnp.float32)
    zqT = jax.lax.dot_general(e, oh_T, (((0,), (0,)), ((), ())),
                              preferred_element_type=jnp.float32)
    zq_ref[0] = zb + (zqT - zb)

    row_iota = jax.lax.broadcasted_iota(jnp.int32, (TILE, SIZE), 1)
    enc = (row_iota == idx[:, None]).astype(jnp.float32)
    enc_ref[...] = enc
    counts_ref[...] += jnp.sum(enc, axis=0, keepdims=True)

    idx_ref[0, 0] = idx

    diff = zqT - zb
    loss_acc_ref[0] += jnp.sum(diff * diff)

    @pl.when(t == N_TILES - 1)
    def _finish():
        loss_ref[0, 0] = BETA_C * loss_acc_ref[0] / (N_ROWS * LATENT_DIM)
        e_mean = counts_ref[...] / N_ROWS
        perp_ref[0, 0] = jnp.exp(-jnp.sum(e_mean * jnp.log(e_mean + 1e-10)))


def kernel(z, embedding_weight):
    z2 = z.reshape(4, 256, 8192)
    zq3, enc, idx3, loss, perp = pl.pallas_call(
        _vq_body,
        grid=(N_TILES,),
        in_specs=[
            pl.BlockSpec((1, 256, TILE), lambda t: (t // TILES_PER_B, 0, t % TILES_PER_B)),
            pl.BlockSpec((SIZE, LATENT_DIM), lambda t: (0, 0)),
        ],
        out_specs=[
            pl.BlockSpec((1, 256, TILE), lambda t: (t // TILES_PER_B, 0, t % TILES_PER_B)),
            pl.BlockSpec((TILE, SIZE), lambda t: (t, 0)),
            pl.BlockSpec((1, 1, TILE), lambda t: (t, 0, 0)),
            pl.BlockSpec((1, 1), lambda t: (0, 0), memory_space=pltpu.SMEM),
            pl.BlockSpec((1, 1), lambda t: (0, 0), memory_space=pltpu.SMEM),
        ],
        out_shape=[
            jax.ShapeDtypeStruct((4, 256, 8192), jnp.float32),
            jax.ShapeDtypeStruct((N_ROWS, SIZE), jnp.float32),
            jax.ShapeDtypeStruct((N_TILES, 1, TILE), jnp.int32),
            jax.ShapeDtypeStruct((1, 1), jnp.float32),
            jax.ShapeDtypeStruct((1, 1), jnp.float32),
        ],
        scratch_shapes=[
            pltpu.VMEM((1, SIZE), jnp.float32),
            pltpu.SMEM((1,), jnp.float32),
        ],
    )(z2, embedding_weight)

    z_q_out = zq3.reshape(4, 256, 8, 32, 32)
    min_idx = idx3.reshape(N_ROWS, 1)
    return (z_q_out, loss[0, 0], perp[0, 0], enc, min_idx)


# speedup vs baseline: 1.2032x; 1.2032x over previous
import jax
import jax.numpy as jnp
from jax.experimental import pallas as pl
from jax.experimental.pallas import tpu as pltpu

SIZE = 1024
LATENT_DIM = 256
BETA_C = 0.25
N_ROWS = 4 * 8 * 32 * 32
TILE = 1024
N_TILES = N_ROWS // TILE
TILES_PER_B = 8192 // TILE


def _vq_body(zb_ref, emb_ref, zq_ref, enc_ref, idx_ref, loss_ref, perp_ref,
             counts_ref, loss_acc_ref, en_ref):
    t = pl.program_id(0)

    e = emb_ref[...]

    @pl.when(t == 0)
    def _init():
        counts_ref[...] = jnp.zeros_like(counts_ref)
        loss_acc_ref[0] = 0.0
        en_ref[...] = jnp.sum(e * e, axis=1, keepdims=True).T

    zb = zb_ref[0]
    en = en_ref[0]

    ez = jax.lax.dot_general(e, zb, (((1,), (0,)), ((), ())),
                             preferred_element_type=jnp.float32)
    s = en[:, None] - 2.0 * ez

    minv = jnp.min(s, axis=0)
    code_iota = jax.lax.broadcasted_iota(jnp.int32, (SIZE, TILE), 0)
    idx = jnp.min(jnp.where(s == minv[None, :], code_iota, SIZE), axis=0)

    oh_T = (code_iota == idx[None, :]).astype(jnp.float32)
    zqT = jax.lax.dot_general(e, oh_T, (((0,), (0,)), ((), ())),
                              preferred_element_type=jnp.float32)
    zq_ref[0] = zb + (zqT - zb)

    row_iota = jax.lax.broadcasted_iota(jnp.int32, (TILE, SIZE), 1)
    enc = (row_iota == idx[:, None]).astype(jnp.float32)
    enc_ref[...] = enc
    counts_ref[...] += jnp.sum(enc, axis=0, keepdims=True)

    idx_ref[0, 0] = idx

    zn = jnp.sum(zb * zb, axis=0)
    loss_acc_ref[0] += jnp.sum(zn + minv)

    @pl.when(t == N_TILES - 1)
    def _finish():
        loss_ref[0, 0] = BETA_C * loss_acc_ref[0] / (N_ROWS * LATENT_DIM)
        e_mean = counts_ref[...] / N_ROWS
        perp_ref[0, 0] = jnp.exp(-jnp.sum(e_mean * jnp.log(e_mean + 1e-10)))


def kernel(z, embedding_weight):
    z2 = z.reshape(4, 256, 8192)
    zq3, enc, idx3, loss, perp = pl.pallas_call(
        _vq_body,
        grid=(N_TILES,),
        in_specs=[
            pl.BlockSpec((1, 256, TILE), lambda t: (t // TILES_PER_B, 0, t % TILES_PER_B)),
            pl.BlockSpec((SIZE, LATENT_DIM), lambda t: (0, 0)),
        ],
        out_specs=[
            pl.BlockSpec((1, 256, TILE), lambda t: (t // TILES_PER_B, 0, t % TILES_PER_B)),
            pl.BlockSpec((TILE, SIZE), lambda t: (t, 0)),
            pl.BlockSpec((1, 1, TILE), lambda t: (t, 0, 0)),
            pl.BlockSpec((1, 1), lambda t: (0, 0), memory_space=pltpu.SMEM),
            pl.BlockSpec((1, 1), lambda t: (0, 0), memory_space=pltpu.SMEM),
        ],
        out_shape=[
            jax.ShapeDtypeStruct((4, 256, 8192), jnp.float32),
            jax.ShapeDtypeStruct((N_ROWS, SIZE), jnp.float32),
            jax.ShapeDtypeStruct((N_TILES, 1, TILE), jnp.int32),
            jax.ShapeDtypeStruct((1, 1), jnp.float32),
            jax.ShapeDtypeStruct((1, 1), jnp.float32),
        ],
        scratch_shapes=[
            pltpu.VMEM((1, SIZE), jnp.float32),
            pltpu.SMEM((1,), jnp.float32),
            pltpu.VMEM((1, SIZE), jnp.float32),
        ],
    )(z2, embedding_weight)

    z_q_out = zq3.reshape(4, 256, 8, 32, 32)
    min_idx = idx3.reshape(N_ROWS, 1)
    return (z_q_out, loss[0, 0], perp[0, 0], enc, min_idx)
